# Initial kernel scaffold; baseline (speedup 1.0000x reference)
import jax
import jax.numpy as jnp
from jax.experimental import pallas as pl


def kernel(users_ids, items_ids, user_table, item_table):
    raise NotImplementedError("write your pallas kernel here")


# speedup vs baseline: 1.1285x; 1.1285x over previous
import functools

import jax
import jax.numpy as jnp
from jax import lax
from jax.experimental import pallas as pl
from jax.experimental.pallas import tpu as pltpu
from jax.experimental.pallas import tpu_sc as plsc

BATCH = 16384
DIM = 128
NC = 2
NS = 16
L = 16
NW = NC * NS
BPW = BATCH // NW
CHUNK = 256
NCHUNK = BPW // CHUNK
GROUPS = CHUNK // L


@functools.partial(
    pl.kernel,
    mesh=plsc.VectorSubcoreMesh(core_axis_name="c", subcore_axis_name="s"),
    out_type=jax.ShapeDtypeStruct((BATCH,), jnp.float32),
    compiler_params=pltpu.CompilerParams(
        needs_layout_passes=False,
        use_tc_tiling_on_sc=False,
    ),
    scratch_types=[
        pltpu.VMEM((BPW,), jnp.int32),
        pltpu.VMEM((BPW,), jnp.int32),
        pltpu.VMEM((CHUNK, DIM), jnp.float32),
        pltpu.VMEM((CHUNK, DIM), jnp.float32),
        pltpu.VMEM((BPW,), jnp.float32),
        pltpu.SemaphoreType.DMA,
        pltpu.SemaphoreType.DMA,
    ],
)
def _sc_dot_kernel(uid_hbm, iid_hbm, ut_hbm, it_hbm, out_hbm,
                   uidx_v, iidx_v, urows_v, irows_v, out_v, sem_u, sem_i):
    wid = lax.axis_index("s") * NC + lax.axis_index("c")
    base = wid * BPW
    pltpu.sync_copy(uid_hbm.at[pl.ds(base, BPW)], uidx_v)
    pltpu.sync_copy(iid_hbm.at[pl.ds(base, BPW)], iidx_v)

    lane = lax.iota(jnp.int32, L)

    def chunk_body(c, carry):
        cu = pltpu.async_copy(
            ut_hbm.at[uidx_v.at[pl.ds(c * CHUNK, CHUNK)]], urows_v, sem_u)
        ci = pltpu.async_copy(
            it_hbm.at[iidx_v.at[pl.ds(c * CHUNK, CHUNK)]], irows_v, sem_i)
        cu.wait()
        ci.wait()

        def group_body(g, carry2):
            out_vec = jnp.zeros((L,), jnp.float32)
            for j in range(L):
                p = g * L + j
                acc = jnp.zeros((L,), jnp.float32)
                for k in range(DIM // L):
                    acc = acc + (urows_v[p, pl.ds(k * L, L)]
                                 * irows_v[p, pl.ds(k * L, L)])
                s = jnp.sum(acc)
                out_vec = jnp.where(lane == j, s, out_vec)
            out_v[pl.ds(c * CHUNK + g * L, L)] = out_vec
            return carry2

        lax.fori_loop(0, GROUPS, group_body, 0)
        return carry

    lax.fori_loop(0, NCHUNK, chunk_body, 0)
    pltpu.sync_copy(out_v, out_hbm.at[pl.ds(base, BPW)])


def kernel(users_ids, items_ids, user_table, item_table):
    return _sc_dot_kernel(users_ids.astype(jnp.int32),
                          items_ids.astype(jnp.int32),
                          user_table, item_table)


# speedup vs baseline: 1.5060x; 1.3345x over previous
import functools

import jax
import jax.numpy as jnp
from jax import lax
from jax.experimental import pallas as pl
from jax.experimental.pallas import tpu as pltpu
from jax.experimental.pallas import tpu_sc as plsc

BATCH = 16384
DIM = 128
NC = 2
NS = 16
L = 16
NW = NC * NS
BPW = BATCH // NW
CHUNK = 128
NCHUNK = BPW // CHUNK
GROUPS = CHUNK // L
NBUF = 2


@functools.partial(
    pl.kernel,
    mesh=plsc.VectorSubcoreMesh(core_axis_name="c", subcore_axis_name="s"),
    out_type=jax.ShapeDtypeStruct((BATCH,), jnp.float32),
    compiler_params=pltpu.CompilerParams(
        needs_layout_passes=False,
        use_tc_tiling_on_sc=False,
    ),
    scratch_types=[
        pltpu.VMEM((BPW,), jnp.int32),
        pltpu.VMEM((BPW,), jnp.int32),
        pltpu.VMEM((NBUF, CHUNK, DIM), jnp.float32),
        pltpu.VMEM((NBUF, CHUNK, DIM), jnp.float32),
        pltpu.VMEM((L * L,), jnp.float32),
        pltpu.VMEM((BPW,), jnp.float32),
        pltpu.SemaphoreType.DMA,
        pltpu.SemaphoreType.DMA,
    ],
)
def _sc_dot_kernel(uid_hbm, iid_hbm, ut_hbm, it_hbm, out_hbm,
                   uidx_v, iidx_v, urows_v, irows_v, part_v, out_v,
                   sem_u, sem_i):
    wid = lax.axis_index("s") * NC + lax.axis_index("c")
    base = wid * BPW
    pltpu.sync_copy(uid_hbm.at[pl.ds(base, BPW)], uidx_v)
    pltpu.sync_copy(iid_hbm.at[pl.ds(base, BPW)], iidx_v)

    lane = lax.iota(jnp.int32, L)

    def fire(c, buf):
        cu = pltpu.async_copy(
            ut_hbm.at[uidx_v.at[pl.ds(c * CHUNK, CHUNK)]], urows_v.at[buf],
            sem_u)
        ci = pltpu.async_copy(
            it_hbm.at[iidx_v.at[pl.ds(c * CHUNK, CHUNK)]], irows_v.at[buf],
            sem_i)
        return cu, ci

    fire(0, 0)

    def chunk_body(c, carry):
        buf = lax.rem(c, NBUF)
        pltpu.make_async_copy(
            ut_hbm.at[uidx_v.at[pl.ds(c * CHUNK, CHUNK)]], urows_v.at[buf],
            sem_u).wait()
        pltpu.make_async_copy(
            it_hbm.at[iidx_v.at[pl.ds(c * CHUNK, CHUNK)]], irows_v.at[buf],
            sem_i).wait()

        @pl.when(c + 1 < NCHUNK)
        def _():
            fire(c + 1, lax.rem(c + 1, NBUF))

        def group_body(g, carry2):
            for j in range(L):
                p = g * L + j
                acc = (urows_v[buf, p, pl.ds(0, L)]
                       * irows_v[buf, p, pl.ds(0, L)])
                for k in range(1, DIM // L):
                    acc = acc + (urows_v[buf, p, pl.ds(k * L, L)]
                                 * irows_v[buf, p, pl.ds(k * L, L)])
                part_v[pl.ds(j * L, L)] = acc
            tot = plsc.load_gather(part_v, [lane * L])
            for t in range(1, L):
                tot = tot + plsc.load_gather(part_v, [lane * L + t])
            out_v[pl.ds(c * CHUNK + g * L, L)] = tot
            return carry2

        lax.fori_loop(0, GROUPS, group_body, 0)
        return carry

    lax.fori_loop(0, NCHUNK, chunk_body, 0)
    pltpu.sync_copy(out_v, out_hbm.at[pl.ds(base, BPW)])


def kernel(users_ids, items_ids, user_table, item_table):
    return _sc_dot_kernel(users_ids.astype(jnp.int32),
                          items_ids.astype(jnp.int32),
                          user_table, item_table)
